# Initial kernel scaffold; baseline (speedup 1.0000x reference)
import jax
import jax.numpy as jnp
from jax.experimental import pallas as pl


def kernel(x, table):
    raise NotImplementedError("write your pallas kernel here")


# speedup vs baseline: 1.0114x; 1.0114x over previous
import functools
import math

import jax
import jax.numpy as jnp
from jax import lax
from jax.experimental import pallas as pl
from jax.experimental.pallas import tpu as pltpu
from jax.experimental.pallas import tpu_sc as plsc

D = 128
L = 16
NC = 2
NS = 16
NW = NC * NS
SUB = 128
C = 256
SCALE = float(math.sqrt(float(D)))


@functools.partial(jax.jit, static_argnums=(2,))
def _gather_scale(idx2, table, B):
    b_per_w = B // NW
    n_chunks = b_per_w // C
    rows_per_chunk = C // SUB

    mesh = plsc.VectorSubcoreMesh(core_axis_name="c", subcore_axis_name="s")

    @functools.partial(
        pl.kernel,
        mesh=mesh,
        out_type=jax.ShapeDtypeStruct((B, D), jnp.float32),
        scratch_types=[
            pltpu.VMEM((rows_per_chunk, SUB), jnp.int32),
            pltpu.VMEM((C, D), jnp.float32),
            pltpu.SemaphoreType.DMA,
        ],
    )
    def k(idx_hbm, table_hbm, out_hbm, idx_v, rows_v, sem):
        wid = lax.axis_index("s") * NC + lax.axis_index("c")
        irow0 = wid * (b_per_w // SUB)

        def chunk_body(g, carry):
            r = irow0 + g * rows_per_chunk
            pltpu.sync_copy(idx_hbm.at[pl.ds(r, rows_per_chunk)], idx_v)
            for j in range(rows_per_chunk):
                pltpu.async_copy(
                    table_hbm.at[idx_v.at[j]],
                    rows_v.at[pl.ds(j * SUB, SUB)],
                    sem,
                ).wait()

            def scale_row(i, c2):
                for v in range(D // L):
                    rows_v[i, pl.ds(v * L, L)] = (
                        rows_v[i, pl.ds(v * L, L)] * SCALE
                    )
                return c2

            lax.fori_loop(0, C, scale_row, 0)
            pltpu.sync_copy(rows_v, out_hbm.at[pl.ds(r * SUB, C)])
            return carry

        lax.fori_loop(0, n_chunks, chunk_body, 0)

    return k(idx2, table)


def kernel(x, table):
    B = x.shape[0] * x.shape[1]
    idx2 = x.reshape(B // SUB, SUB)
    out = _gather_scale(idx2, table, B)
    return out.reshape(x.shape[0], x.shape[1], D)


# speedup vs baseline: 1.7661x; 1.7462x over previous
import functools
import math

import jax
import jax.numpy as jnp
from jax import lax
from jax.experimental import pallas as pl
from jax.experimental.pallas import tpu as pltpu
from jax.experimental.pallas import tpu_sc as plsc

D = 128
L = 16
NC = 2
NS = 16
NW = NC * NS
C = 128
NBUF = 5
SCALE = float(math.sqrt(float(D)))


@functools.partial(jax.jit, static_argnums=(2,))
def _gather_scale(idx2, table, B):
    b_per_w = B // NW
    n_chunks = b_per_w // C
    assert n_chunks % NBUF == 0

    mesh = plsc.VectorSubcoreMesh(core_axis_name="c", subcore_axis_name="s")

    scratch = [pltpu.VMEM((n_chunks, C), jnp.int32)]
    scratch += [pltpu.VMEM((C, D), jnp.float32) for _ in range(NBUF)]
    scratch += [pltpu.SemaphoreType.DMA for _ in range(2 * NBUF)]

    @functools.partial(
        pl.kernel,
        mesh=mesh,
        out_type=jax.ShapeDtypeStruct((B, D), jnp.float32),
        scratch_types=scratch,
    )
    def k(idx_hbm, table_hbm, out_hbm, idx_v, *bufs_and_sems):
        rows = bufs_and_sems[:NBUF]
        gsem = bufs_and_sems[NBUF:2 * NBUF]
        ssem = bufs_and_sems[2 * NBUF:]

        wid = lax.axis_index("s") * NC + lax.axis_index("c")
        orow0 = wid * b_per_w

        pltpu.sync_copy(idx_hbm.at[wid], idx_v)

        def gather(g, b):
            return pltpu.make_async_copy(
                table_hbm.at[idx_v.at[g]], rows[b], gsem[b])

        def store(g, b):
            return pltpu.make_async_copy(
                rows[b], out_hbm.at[pl.ds(orow0 + g * C, C)], ssem[b])

        for b in range(NBUF):
            gather(b, b).start()

        def round_body(go, carry):
            for b in range(NBUF):
                g = go * NBUF + b
                gather(g, b).wait()

                def scale_pair(i, c2):
                    for r in range(2):
                        for v in range(D // L):
                            rows[b][i * 2 + r, pl.ds(v * L, L)] = (
                                rows[b][i * 2 + r, pl.ds(v * L, L)] * SCALE
                            )
                    return c2

                lax.fori_loop(0, C // 2, scale_pair, 0)
                store(g, b).start()

                @pl.when(g + NBUF < n_chunks)
                def _():
                    store(g, b).wait()
                    gather(g + NBUF, b).start()
            return carry

        lax.fori_loop(0, n_chunks // NBUF, round_body, 0)

        for b in range(NBUF):
            store(n_chunks - NBUF + b, b).wait()

    return k(idx2, table)


def kernel(x, table):
    B = x.shape[0] * x.shape[1]
    idx2 = x.reshape(NW, B // (NW * C), C)
    out = _gather_scale(idx2, table, B)
    return out.reshape(x.shape[0], x.shape[1], D)
